# baseline (speedup 1.0000x reference)
import functools

import jax
import jax.numpy as jnp
from jax import lax
from jax.experimental import pallas as pl
from jax.experimental.pallas import tpu as pltpu
from jax.experimental.pallas import tpu_sc as plsc

_DIM = 64
_ROWS = 40960
_NEG = 5
_BATCH = 4096

_NC = 2
_NS = 16
_NW = _NC * _NS
_G = 16
_RPW = _ROWS // _NW
_CH = _RPW // _G


def _sc_scores(comb, u2, v2):
  mesh = plsc.VectorSubcoreMesh(core_axis_name="c", subcore_axis_name="s")

  @functools.partial(
      pl.kernel,
      out_type=[
          jax.ShapeDtypeStruct((_ROWS,), jnp.float32),
          jax.ShapeDtypeStruct((_ROWS,), jnp.float32),
      ],
      mesh=mesh,
      compiler_params=pltpu.CompilerParams(needs_layout_passes=False),
      scratch_types=[
          pltpu.VMEM((2, 128), jnp.int32),
          pltpu.VMEM((2, 128), jnp.int32),
          pltpu.VMEM((2 * _G, 128), jnp.float32),
          pltpu.VMEM((2 * _G, 128), jnp.float32),
          pltpu.VMEM((10 * _G, 128), jnp.float32),
          pltpu.VMEM((16, 6 * _G), jnp.float32),
          pltpu.VMEM((_RPW,), jnp.float32),
          pltpu.VMEM((_RPW,), jnp.float32),
          pltpu.SemaphoreType.DMA,
          pltpu.SemaphoreType.DMA,
          pltpu.SemaphoreType.DMA,
          pltpu.SemaphoreType.DMA,
      ],
  )
  def k(comb_hbm, u_hbm, v_hbm, score_hbm, negsum_hbm,
        idxs, half, u_rows, v_rows, n_rows, partt, score_all,
        negsum, sem_u, sem_v, sem_n0, sem_n1):
    wid = lax.axis_index("s") * _NC + lax.axis_index("c")
    lanes = lax.iota(jnp.int32, 16)

    def chunk(c, carry):
      pltpu.sync_copy(comb_hbm.at[wid, c], idxs)
      for rr in range(2):
        for t in range(8):
          half[rr, pl.ds(16 * t, 16)] = (
              lax.shift_right_logical(idxs[rr, pl.ds(16 * t, 16)], 1))
      cu = pltpu.async_copy(u_hbm.at[half.at[0, pl.ds(0, 2 * _G)]],
                            u_rows, sem_u)
      cv = pltpu.async_copy(v_hbm.at[half.at[0, pl.ds(2 * _G, 2 * _G)]],
                            v_rows, sem_v)
      cn0 = pltpu.async_copy(v_hbm.at[half.at[0, pl.ds(4 * _G, 4 * _G)]],
                             n_rows.at[pl.ds(0, 4 * _G)], sem_n0)
      cn1 = pltpu.async_copy(v_hbm.at[half.at[1, pl.ds(0, 6 * _G)]],
                             n_rows.at[pl.ds(4 * _G, 6 * _G)], sem_n1)
      cu.wait()
      cv.wait()
      cn0.wait()
      cn1.wait()
      offv = [(idxs[rr, pl.ds(16 * t, 16)] & 1) * 64
              for rr in range(2) for t in range(8)]

      def off(p):
        return offv[p // 16][p % 16]

      for i in range(_G):
        pu0 = off(2 * i)
        pu1 = off(2 * i + 1)
        pv0 = off(2 * _G + 2 * i)
        pv1 = off(2 * _G + 2 * i + 1)
        su = [u_rows[2 * i, pl.ds(pu0 + 16 * t, 16)]
              + u_rows[2 * i + 1, pl.ds(pu1 + 16 * t, 16)]
              for t in range(4)]
        p = su[0] * (v_rows[2 * i, pl.ds(pv0, 16)]
                     + v_rows[2 * i + 1, pl.ds(pv1, 16)])
        for t in range(1, 4):
          p = p + su[t] * (v_rows[2 * i, pl.ds(pv0 + 16 * t, 16)]
                           + v_rows[2 * i + 1, pl.ds(pv1 + 16 * t, 16)])
        plsc.store_scatter(partt, [lanes, jnp.full((16,), i, jnp.int32)], p)
        for kk in range(_NEG):
          r = 10 * i + 2 * kk
          pn0 = off(4 * _G + r)
          pn1 = off(4 * _G + r + 1)
          pn = su[0] * (n_rows[r, pl.ds(pn0, 16)]
                        + n_rows[r + 1, pl.ds(pn1, 16)])
          for t in range(1, 4):
            pn = pn + su[t] * (n_rows[r, pl.ds(pn0 + 16 * t, 16)]
                               + n_rows[r + 1, pl.ds(pn1 + 16 * t, 16)])
          plsc.store_scatter(
              partt, [lanes, jnp.full((16,), (1 + kk) * _G + i, jnp.int32)],
              pn)
      acc = []
      for q in range(1 + _NEG):
        a = partt[0, pl.ds(q * _G, _G)]
        for t in range(1, 16):
          a = a + partt[t, pl.ds(q * _G, _G)]
        acc.append(a)
      ds = pl.ds(c * _G, _G)
      score_all[ds] = acc[0] * 0.25
      s = jnp.exp(acc[1] * 0.25)
      for kk in range(2, 1 + _NEG):
        s = s + jnp.exp(acc[kk] * 0.25)
      negsum[ds] = s
      return carry

    lax.fori_loop(0, _CH, chunk, 0)

    pltpu.sync_copy(score_all, score_hbm.at[pl.ds(wid * _RPW, _RPW)])
    pltpu.sync_copy(negsum, negsum_hbm.at[pl.ds(wid * _RPW, _RPW)])

  return k(comb, u2, v2)


def _tc_loss(score2d, negsum2d):
  def body(s_ref, n_ref, o_ref):
    val = (jnp.sum(jnp.log(1.0 + n_ref[...]))
           - jnp.sum(s_ref[...])) * (1.0 / _BATCH)
    o_ref[...] = jnp.broadcast_to(val, (1, 1))

  return pl.pallas_call(
      body,
      out_shape=jax.ShapeDtypeStruct((1, 1), jnp.float32),
  )(score2d, negsum2d)


def kernel(pos_u, pos_v, neg_v, u_weight, v_weight):
  pu_r = pos_u.reshape(_NW, _CH, 2 * _G)
  pv_r = pos_v.reshape(_NW, _CH, 2 * _G)
  nv_r = neg_v.reshape(_NW, _CH, 10 * _G)
  pad = jnp.zeros((_NW, _CH, 2 * _G), jnp.int32)
  comb = jnp.concatenate([pu_r, pv_r, nv_r, pad], axis=2)
  comb = comb.reshape(_NW, _CH, 2, 128)
  u2 = u_weight.reshape(-1, 128)
  v2 = v_weight.reshape(-1, 128)
  score, negsum = _sc_scores(comb, u2, v2)
  loss = _tc_loss(score.reshape(_ROWS // 128, 128),
                  negsum.reshape(_ROWS // 128, 128))
  return loss[0, 0]


# speedup vs baseline: 1.1696x; 1.1696x over previous
import functools

import jax
import jax.numpy as jnp
from jax import lax
from jax.experimental import pallas as pl
from jax.experimental.pallas import tpu as pltpu
from jax.experimental.pallas import tpu_sc as plsc

_DIM = 64
_ROWS = 40960
_NEG = 5
_BATCH = 4096

_NC = 2
_NS = 16
_NW = _NC * _NS
_G = 16
_RPW = _ROWS // _NW
_CH = _RPW // _G
_NPW = _RPW * _NEG


def _sc_scores(pu, pv, nv, u2, v2):
  mesh = plsc.VectorSubcoreMesh(core_axis_name="c", subcore_axis_name="s")

  @functools.partial(
      pl.kernel,
      out_type=[
          jax.ShapeDtypeStruct((_ROWS,), jnp.float32),
          jax.ShapeDtypeStruct((_ROWS,), jnp.float32),
      ],
      mesh=mesh,
      compiler_params=pltpu.CompilerParams(needs_layout_passes=False),
      scratch_types=[
          pltpu.VMEM((_RPW,), jnp.int32),
          pltpu.VMEM((_RPW,), jnp.int32),
          pltpu.VMEM((_RPW,), jnp.int32),
          pltpu.VMEM((_RPW,), jnp.int32),
          pltpu.VMEM((_NPW,), jnp.int32),
          pltpu.VMEM((_NPW,), jnp.int32),
          pltpu.VMEM((_RPW,), jnp.int32),
          pltpu.VMEM((_RPW,), jnp.int32),
          pltpu.VMEM((_RPW,), jnp.int32),
          pltpu.VMEM((_RPW,), jnp.int32),
          pltpu.VMEM((_NPW,), jnp.int32),
          pltpu.VMEM((_NPW,), jnp.int32),
          pltpu.VMEM((2, _G, 128), jnp.float32),
          pltpu.VMEM((2, _G, 128), jnp.float32),
          pltpu.VMEM((2, _G, 128), jnp.float32),
          pltpu.VMEM((2, _G, 128), jnp.float32),
          pltpu.VMEM((2, _NEG * _G, 128), jnp.float32),
          pltpu.VMEM((2, _NEG * _G, 128), jnp.float32),
          pltpu.VMEM((16, 6 * _G), jnp.float32),
          pltpu.VMEM((_RPW,), jnp.float32),
          pltpu.VMEM((_RPW,), jnp.float32),
          pltpu.SemaphoreType.DMA((2,)),
          pltpu.SemaphoreType.DMA((2,)),
          pltpu.SemaphoreType.DMA((2,)),
          pltpu.SemaphoreType.DMA((2,)),
          pltpu.SemaphoreType.DMA((2,)),
          pltpu.SemaphoreType.DMA((2,)),
      ],
  )
  def k(pu_hbm, pv_hbm, nv_hbm, u_hbm, v_hbm, score_hbm, negsum_hbm,
        iu0, iu1, iv0, iv1, in0, in1,
        hu0, hu1, hv0, hv1, hn0, hn1,
        gu0, gu1, gv0, gv1, gn0, gn1,
        partt, score_all, negsum,
        su0, su1, sv0, sv1, sn0, sn1):
    wid = lax.axis_index("s") * _NC + lax.axis_index("c")
    lanes = lax.iota(jnp.int32, 16)
    base = wid * _RPW
    nbase = wid * _NPW

    pltpu.sync_copy(pu_hbm.at[pl.ds(base, _RPW)], iu0)
    pltpu.sync_copy(pu_hbm.at[pl.ds(_ROWS + base, _RPW)], iu1)
    pltpu.sync_copy(pv_hbm.at[pl.ds(base, _RPW)], iv0)
    pltpu.sync_copy(pv_hbm.at[pl.ds(_ROWS + base, _RPW)], iv1)
    pltpu.sync_copy(nv_hbm.at[pl.ds(nbase, _NPW)], in0)
    pltpu.sync_copy(nv_hbm.at[pl.ds(_ROWS * _NEG + nbase, _NPW)], in1)

    def halve(i, carry):
      d = pl.ds(i * 16, 16)
      hu0[d] = lax.shift_right_logical(iu0[d], 1)
      hu1[d] = lax.shift_right_logical(iu1[d], 1)
      hv0[d] = lax.shift_right_logical(iv0[d], 1)
      hv1[d] = lax.shift_right_logical(iv1[d], 1)
      return carry

    lax.fori_loop(0, _RPW // 16, halve, 0)

    def halven(i, carry):
      d = pl.ds(i * 16, 16)
      hn0[d] = lax.shift_right_logical(in0[d], 1)
      hn1[d] = lax.shift_right_logical(in1[d], 1)
      return carry

    lax.fori_loop(0, _NPW // 16, halven, 0)

    def copies(c, s):
      du = pl.ds(c * _G, _G)
      dn = pl.ds(c * _NEG * _G, _NEG * _G)
      return [
          pltpu.make_async_copy(u_hbm.at[hu0.at[du]], gu0.at[s], su0.at[s]),
          pltpu.make_async_copy(u_hbm.at[hu1.at[du]], gu1.at[s], su1.at[s]),
          pltpu.make_async_copy(v_hbm.at[hv0.at[du]], gv0.at[s], sv0.at[s]),
          pltpu.make_async_copy(v_hbm.at[hv1.at[du]], gv1.at[s], sv1.at[s]),
          pltpu.make_async_copy(v_hbm.at[hn0.at[dn]], gn0.at[s], sn0.at[s]),
          pltpu.make_async_copy(v_hbm.at[hn1.at[dn]], gn1.at[s], sn1.at[s]),
      ]

    for cp in copies(0, 0):
      cp.start()

    def chunk(c, carry):
      s = lax.rem(c, 2)

      @pl.when(c + 1 < _CH)
      def _():
        for cp in copies(c + 1, 1 - s):
          cp.start()

      for cp in copies(c, s):
        cp.wait()

      du = pl.ds(c * _G, _G)
      ou0 = (iu0[du] & 1) * 64
      ou1 = (iu1[du] & 1) * 64
      ov0 = (iv0[du] & 1) * 64
      ov1 = (iv1[du] & 1) * 64
      on0 = [(in0[pl.ds(c * _NEG * _G + 16 * t, 16)] & 1) * 64
             for t in range(_NEG)]
      on1 = [(in1[pl.ds(c * _NEG * _G + 16 * t, 16)] & 1) * 64
             for t in range(_NEG)]

      for i in range(_G):
        a0 = ou0[i]
        a1 = ou1[i]
        b0 = ov0[i]
        b1 = ov1[i]
        su = [gu0[s, i, pl.ds(a0 + 16 * t, 16)]
              + gu1[s, i, pl.ds(a1 + 16 * t, 16)]
              for t in range(4)]
        p = su[0] * (gv0[s, i, pl.ds(b0, 16)] + gv1[s, i, pl.ds(b1, 16)])
        for t in range(1, 4):
          p = p + su[t] * (gv0[s, i, pl.ds(b0 + 16 * t, 16)]
                           + gv1[s, i, pl.ds(b1 + 16 * t, 16)])
        plsc.store_scatter(partt, [lanes, jnp.full((16,), i, jnp.int32)], p)
        for kk in range(_NEG):
          r = _NEG * i + kk
          c0 = on0[r // 16][r % 16]
          c1 = on1[r // 16][r % 16]
          pn = su[0] * (gn0[s, r, pl.ds(c0, 16)]
                        + gn1[s, r, pl.ds(c1, 16)])
          for t in range(1, 4):
            pn = pn + su[t] * (gn0[s, r, pl.ds(c0 + 16 * t, 16)]
                               + gn1[s, r, pl.ds(c1 + 16 * t, 16)])
          plsc.store_scatter(
              partt, [lanes, jnp.full((16,), (1 + kk) * _G + i, jnp.int32)],
              pn)

      acc = []
      for q in range(1 + _NEG):
        a = partt[0, pl.ds(q * _G, _G)]
        for t in range(1, 16):
          a = a + partt[t, pl.ds(q * _G, _G)]
        acc.append(a)
      ds = pl.ds(c * _G, _G)
      score_all[ds] = acc[0] * 0.25
      e = jnp.exp(acc[1] * 0.25)
      for kk in range(2, 1 + _NEG):
        e = e + jnp.exp(acc[kk] * 0.25)
      negsum[ds] = e
      return carry

    lax.fori_loop(0, _CH, chunk, 0)

    pltpu.sync_copy(score_all, score_hbm.at[pl.ds(wid * _RPW, _RPW)])
    pltpu.sync_copy(negsum, negsum_hbm.at[pl.ds(wid * _RPW, _RPW)])

  return k(pu, pv, nv, u2, v2)


def _tc_loss(score2d, negsum2d):
  def body(s_ref, n_ref, o_ref):
    val = (jnp.sum(jnp.log(1.0 + n_ref[...]))
           - jnp.sum(s_ref[...])) * (1.0 / _BATCH)
    o_ref[...] = jnp.broadcast_to(val, (1, 1))

  return pl.pallas_call(
      body,
      out_shape=jax.ShapeDtypeStruct((1, 1), jnp.float32),
  )(score2d, negsum2d)


def kernel(pos_u, pos_v, neg_v, u_weight, v_weight):
  pu = jnp.transpose(pos_u).reshape(-1)
  pv = jnp.transpose(pos_v).reshape(-1)
  nv = jnp.transpose(neg_v).reshape(-1)
  u2 = u_weight.reshape(-1, 128)
  v2 = v_weight.reshape(-1, 128)
  score, negsum = _sc_scores(pu, pv, nv, u2, v2)
  loss = _tc_loss(score.reshape(_ROWS // 128, 128),
                  negsum.reshape(_ROWS // 128, 128))
  return loss[0, 0]
